# baseline (speedup 1.0000x reference)
import functools

import jax
import jax.numpy as jnp
from jax import lax
from jax.experimental import pallas as pl
from jax.experimental.pallas import tpu as pltpu
from jax.experimental.pallas import tpu_sc as plsc

_N_E = 8192
_E_DIM = 256
_BETA = 0.25

_BT = 512
_NEB = 512


_SUB = 64


def _argmin_body(x_ref, emb_ref, xn_ref, en_ref, idx_out, dmin_out,
                 minval, minidx):
    n = pl.program_id(1)
    nne = pl.num_programs(1)
    eb = emb_ref[pl.ds(n * _NEB, _NEB), :]
    m_t = lax.dot_general(eb, x_ref[...], (((1,), (1,)), ((), ())),
                          preferred_element_type=jnp.float32)
    en_col = en_ref[pl.ds(n * _NEB, _NEB), :]
    xn_row = xn_ref[...]
    big = jnp.float32(2**24)
    iota_sub = lax.broadcasted_iota(jnp.int32, (_SUB, _BT), 0).astype(jnp.float32)
    for c in range(_NEB // _SUB):
        m_c = lax.slice(m_t, (c * _SUB, 0), ((c + 1) * _SUB, _BT))
        en_c = lax.slice(en_col, (c * _SUB, 0), ((c + 1) * _SUB, 1))
        d_c = (xn_row + en_c) - m_c * 2.0
        lmin_c = jnp.min(d_c, axis=0, keepdims=True)
        base = (n * _NEB + c * _SUB).astype(jnp.float32)
        lidx_c = jnp.min(jnp.where(d_c == lmin_c, iota_sub + base, big),
                         axis=0, keepdims=True)
        if c == 0:
            first = n == 0
            upd = jnp.logical_or(first, lmin_c < minval[...])
        else:
            upd = lmin_c < minval[...]
        minidx[...] = jnp.where(upd, lidx_c, minidx[...])
        minval[...] = jnp.where(upd, lmin_c, minval[...])

    @pl.when(n == nne - 1)
    def _():
        idx_out[...] = minidx[...][0].astype(jnp.int32)
        dmin_out[...] = minval[...][0]


def _distance_argmin(latent, emb, xn2, en2):
    nt, nne = 16384 // _BT, _N_E // _NEB
    return pl.pallas_call(
        _argmin_body,
        grid=(nt, nne),
        in_specs=[
            pl.BlockSpec((_BT, _E_DIM), lambda t, n: (t, 0)),
            pl.BlockSpec((_N_E, _E_DIM), lambda t, n: (0, 0)),
            pl.BlockSpec((1, _BT), lambda t, n: (0, t)),
            pl.BlockSpec((_N_E, 1), lambda t, n: (0, 0)),
        ],
        out_specs=[
            pl.BlockSpec((_BT,), lambda t, n: (t,)),
            pl.BlockSpec((_BT,), lambda t, n: (t,)),
        ],
        out_shape=[
            jax.ShapeDtypeStruct((16384,), jnp.int32),
            jax.ShapeDtypeStruct((16384,), jnp.float32),
        ],
        scratch_shapes=[
            pltpu.VMEM((1, _BT), jnp.float32),
            pltpu.VMEM((1, _BT), jnp.float32),
        ],
        compiler_params=pltpu.CompilerParams(
            dimension_semantics=("arbitrary", "arbitrary"),
        ),
    )(latent, emb, xn2, en2)


def _sc_gather(table, indices):
    info = plsc.get_sparse_core_info()
    nc, ns = info.num_cores, info.num_subcores
    nw = nc * ns
    b = 16384
    b_per_w = b // nw
    ch = 128
    nch = b_per_w // ch
    mesh = plsc.VectorSubcoreMesh(core_axis_name="c", subcore_axis_name="s")

    @functools.partial(
        pl.kernel,
        mesh=mesh,
        out_type=jax.ShapeDtypeStruct((b, _E_DIM), jnp.float32),
        scratch_types=[
            pltpu.VMEM((ch,), jnp.int32),
            pltpu.VMEM((ch, _E_DIM), jnp.float32),
            pltpu.SemaphoreType.DMA,
        ],
    )
    def gather_kernel(table_hbm, idx_hbm, out_hbm, idx_v, rows_v, sem):
        wid = lax.axis_index("s") * nc + lax.axis_index("c")
        for c in range(nch):
            base = wid * b_per_w + c * ch
            pltpu.sync_copy(idx_hbm.at[pl.ds(base, ch)], idx_v)
            pltpu.async_copy(table_hbm.at[idx_v], rows_v, sem).wait()
            pltpu.sync_copy(rows_v, out_hbm.at[pl.ds(base, ch)])

    return gather_kernel(table, indices)


def kernel(x, label, idx, emb):
    latent = x.reshape(-1, _E_DIM)
    xn = jnp.sum(latent ** 2, axis=1)
    en = jnp.sum(emb ** 2, axis=1)
    indices, dmin = _distance_argmin(latent, emb, xn[None, :], en[:, None])
    x_q = _sc_gather(emb, indices)
    loss = jnp.sum(dmin) * jnp.float32(_BETA / (16384 * _E_DIM))
    return (x_q.reshape(x.shape), loss, indices.reshape(x.shape[:-1]))


# speedup vs baseline: 1.1302x; 1.1302x over previous
import functools

import jax
import jax.numpy as jnp
from jax import lax
from jax.experimental import pallas as pl
from jax.experimental.pallas import tpu as pltpu
from jax.experimental.pallas import tpu_sc as plsc

_N_E = 8192
_E_DIM = 256
_BETA = 0.25

_BT = 512
_NEB = 512


_SUB = 8


def _argmin_body(x_ref, emb2_ref, xn_ref, en_ref, idx_out, dmin_out,
                 accv_ref, acci_ref):
    n = pl.program_id(1)
    nne = pl.num_programs(1)
    eb = emb2_ref[pl.ds(n * _NEB, _NEB), :]
    m2_t = lax.dot_general(eb, x_ref[...], (((1,), (1,)), ((), ())),
                           preferred_element_type=jnp.float32)
    en_col = en_ref[pl.ds(n * _NEB, _NEB), :]
    xn_row = xn_ref[...]
    iota8 = lax.broadcasted_iota(jnp.int32, (_SUB, _BT), 0).astype(jnp.float32)

    inf = jnp.full((_SUB, _BT), jnp.inf, jnp.float32)
    zero = jnp.zeros((_SUB, _BT), jnp.float32)
    av = jnp.where(n == 0, inf, accv_ref[...])
    ai = jnp.where(n == 0, zero, acci_ref[...])
    for c in range(_NEB // _SUB):
        m_c = lax.slice(m2_t, (c * _SUB, 0), ((c + 1) * _SUB, _BT))
        en_c = lax.slice(en_col, (c * _SUB, 0), ((c + 1) * _SUB, 1))
        d_c = (xn_row + en_c) - m_c
        i_c = iota8 + (n * _NEB + c * _SUB).astype(jnp.float32)
        upd = d_c < av
        av = jnp.where(upd, d_c, av)
        ai = jnp.where(upd, i_c, ai)
    accv_ref[...] = av
    acci_ref[...] = ai

    @pl.when(n == nne - 1)
    def _():
        vs = [lax.slice(av, (r, 0), (r + 1, _BT)) for r in range(_SUB)]
        is_ = [lax.slice(ai, (r, 0), (r + 1, _BT)) for r in range(_SUB)]
        while len(vs) > 1:
            nv, ni = [], []
            for a in range(0, len(vs), 2):
                va, vb = vs[a], vs[a + 1]
                ia, ib = is_[a], is_[a + 1]
                keep_a = jnp.logical_or(va < vb,
                                        jnp.logical_and(va == vb, ia < ib))
                nv.append(jnp.where(keep_a, va, vb))
                ni.append(jnp.where(keep_a, ia, ib))
            vs, is_ = nv, ni
        idx_out[...] = is_[0][0].astype(jnp.int32)
        dmin_out[...] = vs[0][0]


def _distance_argmin(latent, emb, xn2, en2):
    nt, nne = 16384 // _BT, _N_E // _NEB
    return pl.pallas_call(
        _argmin_body,
        grid=(nt, nne),
        in_specs=[
            pl.BlockSpec((_BT, _E_DIM), lambda t, n: (t, 0)),
            pl.BlockSpec((_N_E, _E_DIM), lambda t, n: (0, 0)),
            pl.BlockSpec((1, _BT), lambda t, n: (0, t)),
            pl.BlockSpec((_N_E, 1), lambda t, n: (0, 0)),
        ],
        out_specs=[
            pl.BlockSpec((_BT,), lambda t, n: (t,)),
            pl.BlockSpec((_BT,), lambda t, n: (t,)),
        ],
        out_shape=[
            jax.ShapeDtypeStruct((16384,), jnp.int32),
            jax.ShapeDtypeStruct((16384,), jnp.float32),
        ],
        scratch_shapes=[
            pltpu.VMEM((_SUB, _BT), jnp.float32),
            pltpu.VMEM((_SUB, _BT), jnp.float32),
        ],
        compiler_params=pltpu.CompilerParams(
            dimension_semantics=("arbitrary", "arbitrary"),
        ),
    )(latent, emb, xn2, en2)


def _sc_gather(table, indices):
    info = plsc.get_sparse_core_info()
    nc, ns = info.num_cores, info.num_subcores
    nw = nc * ns
    b = 16384
    b_per_w = b // nw
    ch = 128
    nch = b_per_w // ch
    mesh = plsc.VectorSubcoreMesh(core_axis_name="c", subcore_axis_name="s")

    @functools.partial(
        pl.kernel,
        mesh=mesh,
        out_type=jax.ShapeDtypeStruct((b, _E_DIM), jnp.float32),
        scratch_types=[
            pltpu.VMEM((ch,), jnp.int32),
            pltpu.VMEM((ch, _E_DIM), jnp.float32),
            pltpu.SemaphoreType.DMA,
        ],
    )
    def gather_kernel(table_hbm, idx_hbm, out_hbm, idx_v, rows_v, sem):
        wid = lax.axis_index("s") * nc + lax.axis_index("c")
        for c in range(nch):
            base = wid * b_per_w + c * ch
            pltpu.sync_copy(idx_hbm.at[pl.ds(base, ch)], idx_v)
            pltpu.async_copy(table_hbm.at[idx_v], rows_v, sem).wait()
            pltpu.sync_copy(rows_v, out_hbm.at[pl.ds(base, ch)])

    return gather_kernel(table, indices)


def kernel(x, label, idx, emb):
    latent = x.reshape(-1, _E_DIM)
    xn = jnp.sum(latent ** 2, axis=1)
    en = jnp.sum(emb ** 2, axis=1)
    indices, dmin = _distance_argmin(latent, emb * 2.0, xn[None, :], en[:, None])
    x_q = _sc_gather(emb, indices)
    loss = jnp.sum(dmin) * jnp.float32(_BETA / (16384 * _E_DIM))
    return (x_q.reshape(x.shape), loss, indices.reshape(x.shape[:-1]))
